# baseline (speedup 1.0000x reference)
import functools

import jax
import jax.numpy as jnp
from jax import lax
from jax.experimental import pallas as pl
from jax.experimental.pallas import tpu as pltpu
from jax.experimental.pallas import tpu_sc as plsc

NC = 2
NS = 16
NW = NC * NS

BATCH = 16384
DIM = 128
NUM_RELS = 1000
B_PER_W = BATCH // NW
CHUNK = 128
N_CHUNKS = B_PER_W // CHUNK


def _sc_gather3(head2, rel2, tail2, ent, rtab):
    mesh = plsc.VectorSubcoreMesh(
        core_axis_name="c", subcore_axis_name="s", num_cores=NC, num_subcores=NS
    )
    out_t = (
        jax.ShapeDtypeStruct((BATCH, DIM), jnp.float32),
        jax.ShapeDtypeStruct((BATCH, DIM), jnp.float32),
        jax.ShapeDtypeStruct((BATCH, DIM), jnp.float32),
    )

    @functools.partial(
        pl.kernel,
        out_type=out_t,
        mesh=mesh,
        scratch_types=[
            pltpu.VMEM((3 * N_CHUNKS, CHUNK), jnp.int32),
            [pltpu.VMEM((CHUNK, DIM), jnp.float32) for _ in range(3)],
            pltpu.VMEM_SHARED((NUM_RELS, DIM), jnp.float32),
            [pltpu.SemaphoreType.DMA for _ in range(3)],
            [pltpu.SemaphoreType.DMA for _ in range(3)],
            pltpu.SemaphoreType.DMA,
            pltpu.SemaphoreType.DMA,
        ],
    )
    def k(head_h, rel_h, tail_h, ent_h, rtab_h, ho, ro, to,
          idx_v, rows, rtab_s, gsems, osems, isem, tsem):
        sid = lax.axis_index("s")
        wid = sid * NC + lax.axis_index("c")
        rbase = wid * N_CHUNKS
        obase = wid * B_PER_W

        JOBS = ((0, None, ho), (2, None, to), (1, rtab_s, ro))

        @pl.when(sid == 0)
        def _():
            pltpu.async_copy(rtab_h, rtab_s, tsem)

        for j, idx_h in enumerate((head_h, rel_h, tail_h)):
            pltpu.async_copy(idx_h.at[pl.ds(rbase, N_CHUNKS)],
                             idx_v.at[pl.ds(j * N_CHUNKS, N_CHUNKS)], isem)
        for j, idx_h in enumerate((head_h, rel_h, tail_h)):
            pltpu.make_async_copy(idx_h.at[pl.ds(rbase, N_CHUNKS)],
                                  idx_v.at[pl.ds(j * N_CHUNKS, N_CHUNKS)], isem).wait()

        @pl.when(sid == 0)
        def _():
            pltpu.make_async_copy(rtab_h, rtab_s, tsem).wait()

        plsc.subcore_barrier()

        def gather_copy(s, cc):
            j, tab, _ = JOBS[s]
            if tab is None:
                tab = ent_h
            return pltpu.make_async_copy(
                tab.at[idx_v.at[j * N_CHUNKS + cc]], rows[s], gsems[s])

        def out_copy(s, cc):
            j, _, out = JOBS[s]
            return pltpu.make_async_copy(
                rows[s], out.at[pl.ds(obase + cc * CHUNK, CHUNK)], osems[s])

        for s in range(3):
            gather_copy(s, 0).start()

        @pl.loop(0, N_CHUNKS)
        def _(cc):
            for s in range(3):
                gather_copy(s, cc).wait()
                out_copy(s, cc).start()
            @pl.when(cc < N_CHUNKS - 1)
            def _():
                for s in range(3):
                    out_copy(s, cc).wait()
                    gather_copy(s, cc + 1).start()

        for s in range(3):
            out_copy(s, N_CHUNKS - 1).wait()

    return k(head2, rel2, tail2, ent, rtab)


@jax.jit
def kernel(head, rel, tail, entity_embedding, rel_embedding):
    head2 = head.astype(jnp.int32).reshape(BATCH // CHUNK, CHUNK)
    rel2 = rel.astype(jnp.int32).reshape(BATCH // CHUNK, CHUNK)
    tail2 = tail.astype(jnp.int32).reshape(BATCH // CHUNK, CHUNK)
    return _sc_gather3(head2, rel2, tail2, entity_embedding, rel_embedding)


# speedup vs baseline: 1.0228x; 1.0228x over previous
import functools

import jax
import jax.numpy as jnp
from jax import lax
from jax.experimental import pallas as pl
from jax.experimental.pallas import tpu as pltpu
from jax.experimental.pallas import tpu_sc as plsc

NC = 2
NS = 16
NW = NC * NS

BATCH = 16384
DIM = 128
NUM_RELS = 1000
B_PER_W = BATCH // NW
CHUNK = 64
N_CHUNKS = B_PER_W // CHUNK


def _sc_gather3(head2, rel2, tail2, ent, rtab):
    mesh = plsc.VectorSubcoreMesh(
        core_axis_name="c", subcore_axis_name="s", num_cores=NC, num_subcores=NS
    )
    out_t = (
        jax.ShapeDtypeStruct((BATCH, DIM), jnp.float32),
        jax.ShapeDtypeStruct((BATCH, DIM), jnp.float32),
        jax.ShapeDtypeStruct((BATCH, DIM), jnp.float32),
    )

    NBUF = 10
    NJOBS = 3 * N_CHUNKS
    ORDER = []
    for cc in range(N_CHUNKS):
        ORDER += [(0, cc), (2, cc), (1, cc)]

    @functools.partial(
        pl.kernel,
        out_type=out_t,
        mesh=mesh,
        scratch_types=[
            pltpu.VMEM((NJOBS, CHUNK), jnp.int32),
            [pltpu.VMEM((CHUNK, DIM), jnp.float32) for _ in range(NBUF)],
            pltpu.VMEM_SHARED((NUM_RELS, DIM), jnp.float32),
            [pltpu.SemaphoreType.DMA for _ in range(NBUF)],
            [pltpu.SemaphoreType.DMA for _ in range(NBUF)],
            pltpu.SemaphoreType.DMA,
            pltpu.SemaphoreType.DMA,
        ],
    )
    def k(head_h, rel_h, tail_h, ent_h, rtab_h, ho, ro, to,
          idx_v, rows, rtab_s, gsems, osems, isem, tsem):
        sid = lax.axis_index("s")
        wid = sid * NC + lax.axis_index("c")
        rbase = wid * N_CHUNKS
        obase = wid * B_PER_W

        outs = (ho, ro, to)

        @pl.when(sid == 0)
        def _():
            pltpu.async_copy(rtab_h, rtab_s, tsem)

        for j, idx_h in enumerate((head_h, rel_h, tail_h)):
            pltpu.async_copy(idx_h.at[pl.ds(rbase, N_CHUNKS)],
                             idx_v.at[pl.ds(j * N_CHUNKS, N_CHUNKS)], isem)
        for j, idx_h in enumerate((head_h, rel_h, tail_h)):
            pltpu.make_async_copy(idx_h.at[pl.ds(rbase, N_CHUNKS)],
                                  idx_v.at[pl.ds(j * N_CHUNKS, N_CHUNKS)], isem).wait()

        @pl.when(sid == 0)
        def _():
            pltpu.make_async_copy(rtab_h, rtab_s, tsem).wait()

        plsc.subcore_barrier()

        def gather(i, b):
            j, cc = ORDER[i]
            tab = rtab_s if j == 1 else ent_h
            pltpu.async_copy(tab.at[idx_v.at[j * N_CHUNKS + cc]], rows[b], gsems[b])

        def out_copy(i, b):
            j, cc = ORDER[i]
            return pltpu.make_async_copy(
                rows[b], outs[j].at[pl.ds(obase + cc * CHUNK, CHUNK)], osems[b])

        for i in range(NBUF):
            gather(i, i)
        for i in range(NJOBS):
            b = i % NBUF
            j, cc = ORDER[i]
            tab = rtab_s if j == 1 else ent_h
            pltpu.make_async_copy(tab.at[idx_v.at[j * N_CHUNKS + cc]], rows[b], gsems[b]).wait()
            out_copy(i, b).start()
            ni = i + NBUF
            if ni < NJOBS:
                out_copy(i, b).wait()
                gather(ni, b)
        for i in range(NJOBS - NBUF, NJOBS):
            out_copy(i, i % NBUF).wait()

    return k(head2, rel2, tail2, ent, rtab)


@jax.jit
def kernel(head, rel, tail, entity_embedding, rel_embedding):
    head2 = head.astype(jnp.int32).reshape(BATCH // CHUNK, CHUNK)
    rel2 = rel.astype(jnp.int32).reshape(BATCH // CHUNK, CHUNK)
    tail2 = tail.astype(jnp.int32).reshape(BATCH // CHUNK, CHUNK)
    return _sc_gather3(head2, rel2, tail2, entity_embedding, rel_embedding)


# speedup vs baseline: 1.0641x; 1.0404x over previous
import functools

import jax
import jax.numpy as jnp
from jax import lax
from jax.experimental import pallas as pl
from jax.experimental.pallas import tpu as pltpu
from jax.experimental.pallas import tpu_sc as plsc

NC = 2
NS = 16
NW = NC * NS

BATCH = 16384
DIM = 128
NUM_RELS = 1000
B_PER_W = BATCH // NW
CHUNK = 256
N_CHUNKS = B_PER_W // CHUNK
NBUF = 3
NJOBS = 3 * N_CHUNKS


def _sc_gather3(head, rel, tail, ent, rtab):
    mesh = plsc.VectorSubcoreMesh(
        core_axis_name="c", subcore_axis_name="s", num_cores=NC, num_subcores=NS
    )
    out_t = (
        jax.ShapeDtypeStruct((BATCH, DIM), jnp.float32),
        jax.ShapeDtypeStruct((BATCH, DIM), jnp.float32),
        jax.ShapeDtypeStruct((BATCH, DIM), jnp.float32),
    )

    ORDER = []
    for cc in range(N_CHUNKS):
        ORDER += [(0, cc), (2, cc), (1, cc)]

    @functools.partial(
        pl.kernel,
        out_type=out_t,
        mesh=mesh,
        scratch_types=[
            pltpu.VMEM((3 * B_PER_W,), jnp.int32),
            [pltpu.VMEM((CHUNK, DIM), jnp.float32) for _ in range(NBUF)],
            pltpu.VMEM_SHARED((NUM_RELS, DIM), jnp.float32),
            [pltpu.SemaphoreType.DMA for _ in range(NBUF)],
            [pltpu.SemaphoreType.DMA for _ in range(NBUF)],
            pltpu.SemaphoreType.DMA,
            pltpu.SemaphoreType.DMA,
        ],
    )
    def k(head_h, rel_h, tail_h, ent_h, rtab_h, ho, ro, to,
          idx_v, rows, rtab_s, gsems, osems, isem, tsem):
        sid = lax.axis_index("s")
        wid = sid * NC + lax.axis_index("c")
        base = wid * B_PER_W
        obase = wid * B_PER_W

        outs = (ho, ro, to)

        @pl.when(sid == 0)
        def _():
            pltpu.async_copy(rtab_h, rtab_s, tsem)

        for j, idx_h in enumerate((head_h, rel_h, tail_h)):
            pltpu.async_copy(idx_h.at[pl.ds(base, B_PER_W)],
                             idx_v.at[pl.ds(j * B_PER_W, B_PER_W)], isem)
        for j, idx_h in enumerate((head_h, rel_h, tail_h)):
            pltpu.make_async_copy(idx_h.at[pl.ds(base, B_PER_W)],
                                  idx_v.at[pl.ds(j * B_PER_W, B_PER_W)], isem).wait()

        @pl.when(sid == 0)
        def _():
            pltpu.make_async_copy(rtab_h, rtab_s, tsem).wait()

        plsc.subcore_barrier()

        def gather_copy(i, b):
            j, cc = ORDER[i]
            tab = rtab_s if j == 1 else ent_h
            return pltpu.make_async_copy(
                tab.at[idx_v.at[pl.ds(j * B_PER_W + cc * CHUNK, CHUNK)]],
                rows[b], gsems[b])

        def out_copy(i, b):
            j, cc = ORDER[i]
            return pltpu.make_async_copy(
                rows[b], outs[j].at[pl.ds(obase + cc * CHUNK, CHUNK)], osems[b])

        for i in range(NBUF):
            gather_copy(i, i).start()
        for i in range(NJOBS):
            b = i % NBUF
            gather_copy(i, b).wait()
            out_copy(i, b).start()
            ni = i + NBUF
            if ni < NJOBS:
                out_copy(i, b).wait()
                gather_copy(ni, b).start()
        for i in range(NJOBS - NBUF, NJOBS):
            out_copy(i, i % NBUF).wait()

    return k(head, rel, tail, ent, rtab)


@jax.jit
def kernel(head, rel, tail, entity_embedding, rel_embedding):
    return _sc_gather3(head.astype(jnp.int32), rel.astype(jnp.int32),
                       tail.astype(jnp.int32), entity_embedding, rel_embedding)


# speedup vs baseline: 1.0773x; 1.0124x over previous
import functools

import jax
import jax.numpy as jnp
from jax import lax
from jax.experimental import pallas as pl
from jax.experimental.pallas import tpu as pltpu
from jax.experimental.pallas import tpu_sc as plsc

NC = 2
NS = 16
NW = NC * NS

BATCH = 16384
DIM = 128
NUM_RELS = 1000
B_PER_W = BATCH // NW
CHUNK = 128
N_CHUNKS = B_PER_W // CHUNK


def _sc_gather3(head2, rel2, tail2, ent, rtab):
    mesh = plsc.VectorSubcoreMesh(
        core_axis_name="c", subcore_axis_name="s", num_cores=NC, num_subcores=NS
    )
    out_t = (
        jax.ShapeDtypeStruct((BATCH, DIM), jnp.float32),
        jax.ShapeDtypeStruct((BATCH, DIM), jnp.float32),
        jax.ShapeDtypeStruct((BATCH, DIM), jnp.float32),
    )

    NBUF = 7
    NJOBS = 3 * N_CHUNKS
    ORDER = []
    for cc in range(N_CHUNKS):
        ORDER += [(0, cc), (2, cc), (1, cc)]

    @functools.partial(
        pl.kernel,
        out_type=out_t,
        mesh=mesh,
        scratch_types=[
            pltpu.VMEM((NJOBS, CHUNK), jnp.int32),
            [pltpu.VMEM((CHUNK, DIM), jnp.float32) for _ in range(NBUF)],
            pltpu.VMEM_SHARED((NUM_RELS, DIM), jnp.float32),
            [pltpu.SemaphoreType.DMA for _ in range(NBUF)],
            [pltpu.SemaphoreType.DMA for _ in range(NBUF)],
            pltpu.SemaphoreType.DMA,
            pltpu.SemaphoreType.DMA,
        ],
    )
    def k(head_h, rel_h, tail_h, ent_h, rtab_h, ho, ro, to,
          idx_v, rows, rtab_s, gsems, osems, isem, tsem):
        sid = lax.axis_index("s")
        wid = sid * NC + lax.axis_index("c")
        rbase = wid * N_CHUNKS
        obase = wid * B_PER_W

        outs = (ho, ro, to)

        @pl.when(sid == 0)
        def _():
            pltpu.async_copy(rtab_h, rtab_s, tsem)

        for j, idx_h in enumerate((head_h, rel_h, tail_h)):
            pltpu.async_copy(idx_h.at[pl.ds(rbase, N_CHUNKS)],
                             idx_v.at[pl.ds(j * N_CHUNKS, N_CHUNKS)], isem)
        for j, idx_h in enumerate((head_h, rel_h, tail_h)):
            pltpu.make_async_copy(idx_h.at[pl.ds(rbase, N_CHUNKS)],
                                  idx_v.at[pl.ds(j * N_CHUNKS, N_CHUNKS)], isem).wait()

        @pl.when(sid == 0)
        def _():
            pltpu.make_async_copy(rtab_h, rtab_s, tsem).wait()

        plsc.subcore_barrier()

        def gather(i, b):
            j, cc = ORDER[i]
            tab = rtab_s if j == 1 else ent_h
            pltpu.async_copy(tab.at[idx_v.at[j * N_CHUNKS + cc]], rows[b], gsems[b])

        def out_copy(i, b):
            j, cc = ORDER[i]
            return pltpu.make_async_copy(
                rows[b], outs[j].at[pl.ds(obase + cc * CHUNK, CHUNK)], osems[b])

        for i in range(NBUF):
            gather(i, i)
        for i in range(NJOBS):
            b = i % NBUF
            j, cc = ORDER[i]
            tab = rtab_s if j == 1 else ent_h
            pltpu.make_async_copy(tab.at[idx_v.at[j * N_CHUNKS + cc]], rows[b], gsems[b]).wait()
            out_copy(i, b).start()
            ni = i + NBUF
            if ni < NJOBS:
                out_copy(i, b).wait()
                gather(ni, b)
        for i in range(NJOBS - NBUF, NJOBS):
            out_copy(i, i % NBUF).wait()

    return k(head2, rel2, tail2, ent, rtab)


@jax.jit
def kernel(head, rel, tail, entity_embedding, rel_embedding):
    head2 = head.astype(jnp.int32).reshape(BATCH // CHUNK, CHUNK)
    rel2 = rel.astype(jnp.int32).reshape(BATCH // CHUNK, CHUNK)
    tail2 = tail.astype(jnp.int32).reshape(BATCH // CHUNK, CHUNK)
    return _sc_gather3(head2, rel2, tail2, entity_embedding, rel_embedding)


# speedup vs baseline: 1.0802x; 1.0026x over previous
import functools

import jax
import jax.numpy as jnp
from jax import lax
from jax.experimental import pallas as pl
from jax.experimental.pallas import tpu as pltpu
from jax.experimental.pallas import tpu_sc as plsc

NC = 2
NS = 16
NW = NC * NS

BATCH = 16384
DIM = 128
NUM_RELS = 1000
B_PER_W = BATCH // NW
CHUNK = 128
N_CHUNKS = B_PER_W // CHUNK


def _sc_gather3(head2, rel2, tail2, ent, rtab):
    mesh = plsc.VectorSubcoreMesh(
        core_axis_name="c", subcore_axis_name="s", num_cores=NC, num_subcores=NS
    )
    out_t = (
        jax.ShapeDtypeStruct((BATCH, DIM), jnp.float32),
        jax.ShapeDtypeStruct((BATCH, DIM), jnp.float32),
        jax.ShapeDtypeStruct((BATCH, DIM), jnp.float32),
    )

    NBUF = 7
    NJOBS = 3 * N_CHUNKS
    ORDER = [(j, cc) for j in (0, 2, 1) for cc in range(N_CHUNKS)]

    @functools.partial(
        pl.kernel,
        out_type=out_t,
        mesh=mesh,
        scratch_types=[
            pltpu.VMEM((NJOBS, CHUNK), jnp.int32),
            [pltpu.VMEM((CHUNK, DIM), jnp.float32) for _ in range(NBUF)],
            pltpu.VMEM_SHARED((NUM_RELS, DIM), jnp.float32),
            [pltpu.SemaphoreType.DMA for _ in range(NBUF)],
            [pltpu.SemaphoreType.DMA for _ in range(NBUF)],
            pltpu.SemaphoreType.DMA,
            pltpu.SemaphoreType.DMA,
        ],
    )
    def k(head_h, rel_h, tail_h, ent_h, rtab_h, ho, ro, to,
          idx_v, rows, rtab_s, gsems, osems, isem, tsem):
        sid = lax.axis_index("s")
        wid = sid * NC + lax.axis_index("c")
        rbase = wid * N_CHUNKS
        obase = wid * B_PER_W

        outs = (ho, ro, to)

        @pl.when(sid == 0)
        def _():
            pltpu.async_copy(rtab_h, rtab_s, tsem)

        for j, idx_h in enumerate((head_h, rel_h, tail_h)):
            pltpu.async_copy(idx_h.at[pl.ds(rbase, N_CHUNKS)],
                             idx_v.at[pl.ds(j * N_CHUNKS, N_CHUNKS)], isem)
        for j, idx_h in enumerate((head_h, rel_h, tail_h)):
            pltpu.make_async_copy(idx_h.at[pl.ds(rbase, N_CHUNKS)],
                                  idx_v.at[pl.ds(j * N_CHUNKS, N_CHUNKS)], isem).wait()

        @pl.when(sid == 0)
        def _():
            pltpu.make_async_copy(rtab_h, rtab_s, tsem).wait()

        plsc.subcore_barrier()

        def gather(i, b):
            j, cc = ORDER[i]
            tab = rtab_s if j == 1 else ent_h
            pltpu.async_copy(tab.at[idx_v.at[j * N_CHUNKS + cc]], rows[b], gsems[b])

        def out_copy(i, b):
            j, cc = ORDER[i]
            return pltpu.make_async_copy(
                rows[b], outs[j].at[pl.ds(obase + cc * CHUNK, CHUNK)], osems[b])

        for i in range(NBUF):
            gather(i, i)
        for i in range(NJOBS):
            b = i % NBUF
            j, cc = ORDER[i]
            tab = rtab_s if j == 1 else ent_h
            pltpu.make_async_copy(tab.at[idx_v.at[j * N_CHUNKS + cc]], rows[b], gsems[b]).wait()
            out_copy(i, b).start()
            ni = i + NBUF
            if ni < NJOBS:
                out_copy(i, b).wait()
                gather(ni, b)
        for i in range(NJOBS - NBUF, NJOBS):
            out_copy(i, i % NBUF).wait()

    return k(head2, rel2, tail2, ent, rtab)


@jax.jit
def kernel(head, rel, tail, entity_embedding, rel_embedding):
    head2 = head.astype(jnp.int32).reshape(BATCH // CHUNK, CHUNK)
    rel2 = rel.astype(jnp.int32).reshape(BATCH // CHUNK, CHUNK)
    tail2 = tail.astype(jnp.int32).reshape(BATCH // CHUNK, CHUNK)
    return _sc_gather3(head2, rel2, tail2, entity_embedding, rel_embedding)
